# baseline (speedup 1.0000x reference)
import functools

import jax
import jax.numpy as jnp
from jax.experimental import pallas as pl
from jax.experimental.pallas import tpu as pltpu

_K = 5
_P = 2


def _block_body(x_ref, s_ref, b_ref, w_hbm, o_ref, ypad_ref, wvm_ref, sems,
                *, NB, D, H, WC, WCo):
    rows = NB * D * H

    copies = [
        pltpu.make_async_copy(
            w_hbm.at[pl.ds(kd * H * WC, _K * WC), pl.ds(_P * WCo, WCo)],
            wvm_ref.at[kd], sems.at[kd])
        for kd in range(_K)
    ]
    for c in copies:
        c.start()

    y = jnp.maximum(x_ref[...] * s_ref[...] + b_ref[...], 0.0)

    ypad_ref[...] = jnp.zeros_like(ypad_ref)
    ypad_ref[:, _P:_P + D, _P:_P + H, :] = (
        y.reshape(NB, D, H, WC).astype(ypad_ref.dtype))

    parts = []
    for kd in range(_K):
        for dh in range(_K):
            parts.append(ypad_ref[:, kd:kd + D, dh:dh + H, :])
    lhs = jnp.concatenate(parts, axis=-1).reshape(rows, _K * _K * WC)

    for c in copies:
        c.wait()

    o_ref[...] = jnp.dot(lhs, wvm_ref[...].reshape(_K * _K * WC, -1),
                         preferred_element_type=jnp.float32)


@jax.jit
def kernel(x, scale_t, bias_t, w_mat):
    N, D, H, W, Cin = x.shape
    WC = W * Cin
    L_out = w_mat.shape[1]
    Cout = L_out // (H * W)
    WCo = W * Cout

    x2 = x.reshape(N * D * H, WC)

    rows = N * D * H

    body = functools.partial(_block_body, NB=N, D=D, H=H, WC=WC, WCo=WCo)

    out = pl.pallas_call(
        body,
        out_shape=jax.ShapeDtypeStruct((rows, WCo), jnp.float32),
        grid_spec=pltpu.PrefetchScalarGridSpec(
            num_scalar_prefetch=0,
            grid=(1,),
            in_specs=[
                pl.BlockSpec((rows, WC), lambda i: (0, 0)),
                pl.BlockSpec((1, WC), lambda i: (0, 0)),
                pl.BlockSpec((1, WC), lambda i: (0, 0)),
                pl.BlockSpec(memory_space=pl.ANY),
            ],
            out_specs=pl.BlockSpec((rows, WCo), lambda i: (0, 0)),
            scratch_shapes=[
                pltpu.VMEM((N, D + 2 * _P, H + 2 * _P, WC), jnp.bfloat16),
                pltpu.VMEM((_K, _K * WC, WCo), jnp.bfloat16),
                pltpu.SemaphoreType.DMA((_K,)),
            ],
        ),
        compiler_params=pltpu.CompilerParams(
            dimension_semantics=("arbitrary",),
            vmem_limit_bytes=64 * 1024 * 1024),
    )(x2, scale_t, bias_t, w_mat)

    return out.reshape(N, D, H, W, Cout)


# speedup vs baseline: 1.0695x; 1.0695x over previous
import functools

import jax
import jax.numpy as jnp
from jax.experimental import pallas as pl
from jax.experimental.pallas import tpu as pltpu

_K = 5
_P = 2


def _block_body(x_ref, s_ref, b_ref, w_ref, o_ref, hcat_ref, *, NB, D, H, WC):
    rows = NB * D * H

    y = jnp.maximum(x_ref[...] * s_ref[...] + b_ref[...], 0.0)
    y4 = y.reshape(NB, D, H, WC).astype(hcat_ref.dtype)

    hcat_ref[...] = jnp.zeros_like(hcat_ref)
    for dh in range(_K):
        a = max(0, _P - dh)
        b = min(H, H + _P - dh)
        hcat_ref[:, _P:_P + D, a:b, dh * WC:(dh + 1) * WC] = (
            y4[:, :, a + dh - _P:b + dh - _P, :])

    parts = [hcat_ref[:, kd:kd + D, :, :] for kd in range(_K)]
    lhs = jnp.concatenate(parts, axis=-1).reshape(rows, _K * _K * WC)

    o_ref[...] = jnp.dot(lhs, w_ref[...].reshape(_K * _K * WC, -1),
                         preferred_element_type=jnp.float32)


@jax.jit
def kernel(x, scale_t, bias_t, w_mat):
    N, D, H, W, Cin = x.shape
    WC = W * Cin
    L_out = w_mat.shape[1]
    Cout = L_out // (H * W)
    WCo = W * Cout

    x2 = x.reshape(N * D * H, WC)
    w3 = w_mat.reshape(_K, H * WC, L_out)

    rows = N * D * H

    body = functools.partial(_block_body, NB=N, D=D, H=H, WC=WC)

    out = pl.pallas_call(
        body,
        out_shape=jax.ShapeDtypeStruct((rows, WCo), jnp.float32),
        grid_spec=pltpu.PrefetchScalarGridSpec(
            num_scalar_prefetch=0,
            grid=(1,),
            in_specs=[
                pl.BlockSpec((rows, WC), lambda i: (0, 0)),
                pl.BlockSpec((1, WC), lambda i: (0, 0)),
                pl.BlockSpec((1, WC), lambda i: (0, 0)),
                pl.BlockSpec((_K, _K * WC, WCo), lambda i: (0, 0, _P)),
            ],
            out_specs=pl.BlockSpec((rows, WCo), lambda i: (0, 0)),
            scratch_shapes=[
                pltpu.VMEM((N, D + 2 * _P, H, _K * WC), jnp.bfloat16),
            ],
        ),
        compiler_params=pltpu.CompilerParams(
            dimension_semantics=("arbitrary",),
            vmem_limit_bytes=64 * 1024 * 1024),
    )(x2, scale_t, bias_t, w3)

    return out.reshape(N, D, H, W, Cout)
